# baseline (device time: 74317 ns/iter reference)
import jax
import jax.numpy as jnp
from jax import lax
from jax.experimental import pallas as pl
from jax.experimental.pallas import tpu as pltpu

N_DEV = 32
N_STEPS = 5
N_LAYERS = 3
MASKS = (1, 3, 4, 8, 16)
N_CHAINS = 4


def kernel(x, Win0, Wout0, Win1, Wout1, Win2, Wout2):
    b, d_in = x.shape
    _, h_dim = Win0.shape

    def body(x_ref, win0_hbm, wout0_hbm, win1_hbm, wout1_hbm, win2_hbm,
             wout2_hbm, out_ref, send_ref, comm_ref, win_vmem, wout_vmem,
             send_sems, recv_sems, load_sems):
        my = lax.axis_index("i")

        barrier_sem = pltpu.get_barrier_semaphore()
        for s in range(N_STEPS):
            partner = my ^ MASKS[s]
            pl.semaphore_signal(
                barrier_sem, inc=1,
                device_id=(partner,), device_id_type=pl.DeviceIdType.MESH,
            )

        win_hbms = [win0_hbm, win1_hbm, win2_hbm]
        wout_hbms = [wout0_hbm, wout1_hbm, wout2_hbm]

        def load_win(l):
            c = pltpu.make_async_copy(
                win_hbms[l], win_vmem.at[l % 2], load_sems.at[2 * l]
            )
            c.start()
            return c

        def load_wout(l):
            c = pltpu.make_async_copy(
                wout_hbms[l], wout_vmem.at[l % 2], load_sems.at[2 * l + 1]
            )
            c.start()
            return c

        loads = {}
        loads["win0"] = load_win(0)
        loads["wout0"] = load_wout(0)
        loads["win1"] = load_win(1)
        loads["wout1"] = load_wout(1)

        q = h_dim // N_CHAINS
        x_cur = x_ref[:, :].astype(jnp.bfloat16)
        for l in range(N_LAYERS):
            loads[f"win{l}"].wait()
            win_bf = win_vmem[l % 2].astype(jnp.bfloat16)

            rdmas = {}

            def issue(s, c):
                idx = l * N_STEPS + s
                partner = my ^ MASKS[s]
                r = pltpu.make_async_remote_copy(
                    src_ref=send_ref.at[c],
                    dst_ref=comm_ref.at[idx, c],
                    send_sem=send_sems.at[idx, c],
                    recv_sem=recv_sems.at[idx, c],
                    device_id=(partner,),
                    device_id_type=pl.DeviceIdType.MESH,
                )
                r.start()
                rdmas[(s, c)] = r

            for c in range(N_CHAINS):
                send_ref[c, :, :] = jnp.dot(
                    x_cur, win_bf[:, c * q:(c + 1) * q],
                    preferred_element_type=jnp.float32,
                ).astype(jnp.bfloat16)
                if l == 0 and c == 0:
                    pl.semaphore_wait(barrier_sem, N_STEPS)
                issue(0, c)

            for s in range(N_STEPS - 1):
                idx = l * N_STEPS + s
                for c in range(N_CHAINS):
                    rdmas[(s, c)].wait()
                send_ref[:, :, :] = send_ref[:, :, :] + comm_ref[idx]
                for c in range(N_CHAINS):
                    issue(s + 1, c)

            idx = l * N_STEPS + N_STEPS - 1
            loads[f"wout{l}"].wait()
            wout_bf = wout_vmem[l % 2].astype(jnp.bfloat16)
            x_f32 = None
            for c in range(N_CHAINS):
                rdmas[(N_STEPS - 1, c)].wait()
                hc = jnp.maximum(send_ref[c, :, :] + comm_ref[idx, c], 0.0)
                xc = jnp.dot(hc, wout_bf[c * q:(c + 1) * q, :],
                             preferred_element_type=jnp.float32)
                x_f32 = xc if x_f32 is None else x_f32 + xc
            x_cur = x_f32.astype(jnp.bfloat16)

            if l + 2 < N_LAYERS:
                loads[f"win{l + 2}"] = load_win(l + 2)
                loads[f"wout{l + 2}"] = load_wout(l + 2)
        out_ref[:, :] = x_f32

    return pl.pallas_call(
        body,
        out_shape=jax.ShapeDtypeStruct((b, d_in), jnp.float32),
        in_specs=[pl.BlockSpec(memory_space=pltpu.VMEM)]
        + [pl.BlockSpec(memory_space=pl.ANY)] * 6,
        out_specs=pl.BlockSpec(memory_space=pltpu.VMEM),
        scratch_shapes=[
            pltpu.VMEM((N_CHAINS, b, h_dim // N_CHAINS), jnp.bfloat16),
            pltpu.VMEM((N_LAYERS * N_STEPS, N_CHAINS, b, h_dim // N_CHAINS), jnp.bfloat16),
            pltpu.VMEM((2, d_in, h_dim), jnp.float32),
            pltpu.VMEM((2, h_dim, d_in), jnp.float32),
            pltpu.SemaphoreType.DMA((N_LAYERS * N_STEPS, N_CHAINS)),
            pltpu.SemaphoreType.DMA((N_LAYERS * N_STEPS, N_CHAINS)),
            pltpu.SemaphoreType.DMA((2 * N_LAYERS,)),
        ],
        compiler_params=pltpu.CompilerParams(collective_id=0),
    )(x, Win0, Wout0, Win1, Wout1, Win2, Wout2)


# device time: 61799 ns/iter; 1.2026x vs baseline; 1.2026x over previous
import jax
import jax.numpy as jnp
from jax import lax
from jax.experimental import pallas as pl
from jax.experimental.pallas import tpu as pltpu

N_DEV = 32
N_STEPS = 5
N_LAYERS = 3
MASKS = (4, 1, 3, 8, 16)
N_CHAINS = 4


def kernel(x, Win0, Wout0, Win1, Wout1, Win2, Wout2):
    b, d_in = x.shape
    _, h_dim = Win0.shape

    def body(x_ref, win0_hbm, wout0_hbm, win1_hbm, wout1_hbm, win2_hbm,
             wout2_hbm, out_ref, send_ref, comm_ref, win_vmem, wout_vmem,
             send_sems, recv_sems, load_sems):
        my = lax.axis_index("i")

        barrier_sem = pltpu.get_barrier_semaphore()
        for s in range(N_STEPS):
            partner = my ^ MASKS[s]
            pl.semaphore_signal(
                barrier_sem, inc=1,
                device_id=(partner,), device_id_type=pl.DeviceIdType.MESH,
            )

        win_hbms = [win0_hbm, win1_hbm, win2_hbm]
        wout_hbms = [wout0_hbm, wout1_hbm, wout2_hbm]

        def load_win(l):
            c = pltpu.make_async_copy(
                win_hbms[l], win_vmem.at[l % 2], load_sems.at[2 * l]
            )
            c.start()
            return c

        def load_wout(l):
            c = pltpu.make_async_copy(
                wout_hbms[l], wout_vmem.at[l % 2], load_sems.at[2 * l + 1]
            )
            c.start()
            return c

        loads = {}
        loads["win0"] = load_win(0)
        loads["wout0"] = load_wout(0)
        loads["win1"] = load_win(1)
        loads["wout1"] = load_wout(1)

        q = h_dim // N_CHAINS
        x_cur = x_ref[:, :].astype(jnp.bfloat16)
        for l in range(N_LAYERS):
            loads[f"win{l}"].wait()
            win_bf = win_vmem[l % 2].astype(jnp.bfloat16)

            rdmas = {}

            def issue(s, c):
                idx = l * N_STEPS + s
                partner = my ^ MASKS[s]
                r = pltpu.make_async_remote_copy(
                    src_ref=send_ref.at[c],
                    dst_ref=comm_ref.at[idx, c],
                    send_sem=send_sems.at[idx, c],
                    recv_sem=recv_sems.at[idx, c],
                    device_id=(partner,),
                    device_id_type=pl.DeviceIdType.MESH,
                )
                r.start()
                rdmas[(s, c)] = r

            for c in range(N_CHAINS):
                send_ref[c, :, :] = jnp.dot(
                    x_cur, win_bf[:, c * q:(c + 1) * q],
                    preferred_element_type=jnp.float32,
                ).astype(jnp.bfloat16)
                if l == 0 and c == 0:
                    pl.semaphore_wait(barrier_sem, N_STEPS)
                issue(0, c)

            for s in range(N_STEPS - 1):
                idx = l * N_STEPS + s
                for c in range(N_CHAINS):
                    rdmas[(s, c)].wait()
                    send_ref[c, :, :] = send_ref[c, :, :] + comm_ref[idx, c]
                    issue(s + 1, c)

            idx = l * N_STEPS + N_STEPS - 1
            loads[f"wout{l}"].wait()
            wout_bf = wout_vmem[l % 2].astype(jnp.bfloat16)
            x_f32 = None
            for c in range(N_CHAINS):
                rdmas[(N_STEPS - 1, c)].wait()
                hc = jnp.maximum(send_ref[c, :, :] + comm_ref[idx, c], 0.0)
                xc = jnp.dot(hc, wout_bf[c * q:(c + 1) * q, :],
                             preferred_element_type=jnp.float32)
                x_f32 = xc if x_f32 is None else x_f32 + xc
            x_cur = x_f32.astype(jnp.bfloat16)

            if l + 2 < N_LAYERS:
                loads[f"win{l + 2}"] = load_win(l + 2)
                loads[f"wout{l + 2}"] = load_wout(l + 2)
        out_ref[:, :] = x_f32

    return pl.pallas_call(
        body,
        out_shape=jax.ShapeDtypeStruct((b, d_in), jnp.float32),
        in_specs=[pl.BlockSpec(memory_space=pltpu.VMEM)]
        + [pl.BlockSpec(memory_space=pl.ANY)] * 6,
        out_specs=pl.BlockSpec(memory_space=pltpu.VMEM),
        scratch_shapes=[
            pltpu.VMEM((N_CHAINS, b, h_dim // N_CHAINS), jnp.bfloat16),
            pltpu.VMEM((N_LAYERS * N_STEPS, N_CHAINS, b, h_dim // N_CHAINS), jnp.bfloat16),
            pltpu.VMEM((2, d_in, h_dim), jnp.float32),
            pltpu.VMEM((2, h_dim, d_in), jnp.float32),
            pltpu.SemaphoreType.DMA((N_LAYERS * N_STEPS, N_CHAINS)),
            pltpu.SemaphoreType.DMA((N_LAYERS * N_STEPS, N_CHAINS)),
            pltpu.SemaphoreType.DMA((2 * N_LAYERS,)),
        ],
        compiler_params=pltpu.CompilerParams(collective_id=0),
    )(x, Win0, Wout0, Win1, Wout1, Win2, Wout2)


# device time: 61641 ns/iter; 1.2056x vs baseline; 1.0026x over previous
import jax
import jax.numpy as jnp
from jax import lax
from jax.experimental import pallas as pl
from jax.experimental.pallas import tpu as pltpu

N_DEV = 32
N_STEPS = 5
N_LAYERS = 3
MASKS = (4, 1, 3, 8, 16)
N_CHAINS = 4


def kernel(x, Win0, Wout0, Win1, Wout1, Win2, Wout2):
    b, d_in = x.shape
    _, h_dim = Win0.shape

    def body(x_ref, win0_hbm, wout0_hbm, win1_hbm, wout1_hbm, win2_hbm,
             wout2_hbm, out_ref, send_ref, comm_ref, win_vmem, wout_vmem,
             send_sems, recv_sems, load_sems):
        my = lax.axis_index("i")

        barrier_sem = pltpu.get_barrier_semaphore()
        for s in range(N_STEPS):
            partner = my ^ MASKS[s]
            pl.semaphore_signal(
                barrier_sem, inc=1,
                device_id=(partner,), device_id_type=pl.DeviceIdType.MESH,
            )

        win_hbms = [win0_hbm, win1_hbm, win2_hbm]
        wout_hbms = [wout0_hbm, wout1_hbm, wout2_hbm]

        def load_win(l):
            c = pltpu.make_async_copy(
                win_hbms[l], win_vmem.at[l % 2], load_sems.at[2 * l]
            )
            c.start()
            return c

        def load_wout(l):
            c = pltpu.make_async_copy(
                wout_hbms[l], wout_vmem.at[l % 2], load_sems.at[2 * l + 1]
            )
            c.start()
            return c

        loads = {}
        loads["win0"] = load_win(0)
        loads["wout0"] = load_wout(0)
        loads["win1"] = load_win(1)
        loads["wout1"] = load_wout(1)

        q = h_dim // N_CHAINS
        x_cur = x_ref[:, :]
        for l in range(N_LAYERS):
            loads[f"win{l}"].wait()
            win_bf = win_vmem[l % 2]

            rdmas = {}

            def issue(s, c):
                idx = l * N_STEPS + s
                partner = my ^ MASKS[s]
                r = pltpu.make_async_remote_copy(
                    src_ref=send_ref.at[c],
                    dst_ref=comm_ref.at[idx, c],
                    send_sem=send_sems.at[idx, c],
                    recv_sem=recv_sems.at[idx, c],
                    device_id=(partner,),
                    device_id_type=pl.DeviceIdType.MESH,
                )
                r.start()
                rdmas[(s, c)] = r

            for c in range(N_CHAINS):
                send_ref[c, :, :] = jnp.dot(
                    x_cur, win_bf[:, c * q:(c + 1) * q],
                    preferred_element_type=jnp.float32,
                ).astype(jnp.bfloat16)
                if l == 0 and c == 0:
                    pl.semaphore_wait(barrier_sem, N_STEPS)
                issue(0, c)

            for s in range(N_STEPS - 1):
                idx = l * N_STEPS + s
                for c in range(N_CHAINS):
                    rdmas[(s, c)].wait()
                    send_ref[c, :, :] = send_ref[c, :, :] + comm_ref[idx, c]
                    issue(s + 1, c)

            idx = l * N_STEPS + N_STEPS - 1
            loads[f"wout{l}"].wait()
            wout_bf = wout_vmem[l % 2]
            x_f32 = None
            for c in range(N_CHAINS):
                rdmas[(N_STEPS - 1, c)].wait()
                hc = jnp.maximum(send_ref[c, :, :] + comm_ref[idx, c], 0.0)
                xc = jnp.dot(hc.astype(jnp.float32), wout_bf[c * q:(c + 1) * q, :],
                             preferred_element_type=jnp.float32)
                x_f32 = xc if x_f32 is None else x_f32 + xc
            x_cur = x_f32

            if l + 2 < N_LAYERS:
                loads[f"win{l + 2}"] = load_win(l + 2)
                loads[f"wout{l + 2}"] = load_wout(l + 2)
        out_ref[:, :] = x_f32

    return pl.pallas_call(
        body,
        out_shape=jax.ShapeDtypeStruct((b, d_in), jnp.float32),
        in_specs=[pl.BlockSpec(memory_space=pltpu.VMEM)]
        + [pl.BlockSpec(memory_space=pl.ANY)] * 6,
        out_specs=pl.BlockSpec(memory_space=pltpu.VMEM),
        scratch_shapes=[
            pltpu.VMEM((N_CHAINS, b, h_dim // N_CHAINS), jnp.bfloat16),
            pltpu.VMEM((N_LAYERS * N_STEPS, N_CHAINS, b, h_dim // N_CHAINS), jnp.bfloat16),
            pltpu.VMEM((2, d_in, h_dim), jnp.float32),
            pltpu.VMEM((2, h_dim, d_in), jnp.float32),
            pltpu.SemaphoreType.DMA((N_LAYERS * N_STEPS, N_CHAINS)),
            pltpu.SemaphoreType.DMA((N_LAYERS * N_STEPS, N_CHAINS)),
            pltpu.SemaphoreType.DMA((2 * N_LAYERS,)),
        ],
        compiler_params=pltpu.CompilerParams(collective_id=0),
    )(x, Win0, Wout0, Win1, Wout1, Win2, Wout2)


# device time: 61437 ns/iter; 1.2096x vs baseline; 1.0033x over previous
import jax
import jax.numpy as jnp
from jax import lax
from jax.experimental import pallas as pl
from jax.experimental.pallas import tpu as pltpu

N_DEV = 32
N_STEPS = 5
N_LAYERS = 3
MASKS = (4, 1, 3, 8, 16)
N_CHAINS = 8


def kernel(x, Win0, Wout0, Win1, Wout1, Win2, Wout2):
    b, d_in = x.shape
    _, h_dim = Win0.shape

    def body(x_ref, win0_hbm, wout0_hbm, win1_hbm, wout1_hbm, win2_hbm,
             wout2_hbm, out_ref, send_ref, comm_ref, win_vmem, wout_vmem,
             send_sems, recv_sems, load_sems):
        my = lax.axis_index("i")

        barrier_sem = pltpu.get_barrier_semaphore()
        for s in range(N_STEPS):
            partner = my ^ MASKS[s]
            pl.semaphore_signal(
                barrier_sem, inc=1,
                device_id=(partner,), device_id_type=pl.DeviceIdType.MESH,
            )

        win_hbms = [win0_hbm, win1_hbm, win2_hbm]
        wout_hbms = [wout0_hbm, wout1_hbm, wout2_hbm]

        def load_win(l):
            c = pltpu.make_async_copy(
                win_hbms[l], win_vmem.at[l % 2], load_sems.at[2 * l]
            )
            c.start()
            return c

        def load_wout(l):
            c = pltpu.make_async_copy(
                wout_hbms[l], wout_vmem.at[l % 2], load_sems.at[2 * l + 1]
            )
            c.start()
            return c

        loads = {}
        loads["win0"] = load_win(0)
        loads["wout0"] = load_wout(0)
        loads["win1"] = load_win(1)
        loads["wout1"] = load_wout(1)

        q = h_dim // N_CHAINS
        x_cur = x_ref[:, :]
        for l in range(N_LAYERS):
            loads[f"win{l}"].wait()
            win_bf = win_vmem[l % 2]

            rdmas = {}

            def issue(s, c):
                idx = l * N_STEPS + s
                partner = my ^ MASKS[s]
                r = pltpu.make_async_remote_copy(
                    src_ref=send_ref.at[c],
                    dst_ref=comm_ref.at[idx, c],
                    send_sem=send_sems.at[idx, c],
                    recv_sem=recv_sems.at[idx, c],
                    device_id=(partner,),
                    device_id_type=pl.DeviceIdType.MESH,
                )
                r.start()
                rdmas[(s, c)] = r

            for c in range(N_CHAINS):
                send_ref[c, :, :] = jnp.dot(
                    x_cur, win_bf[:, c * q:(c + 1) * q],
                    preferred_element_type=jnp.float32,
                ).astype(jnp.bfloat16)
                if l == 0 and c == 0:
                    pl.semaphore_wait(barrier_sem, N_STEPS)
                issue(0, c)

            for s in range(N_STEPS - 1):
                idx = l * N_STEPS + s
                for c in range(N_CHAINS):
                    rdmas[(s, c)].wait()
                    send_ref[c, :, :] = send_ref[c, :, :] + comm_ref[idx, c]
                    issue(s + 1, c)

            idx = l * N_STEPS + N_STEPS - 1
            loads[f"wout{l}"].wait()
            wout_bf = wout_vmem[l % 2]
            x_f32 = None
            for c in range(N_CHAINS):
                rdmas[(N_STEPS - 1, c)].wait()
                hc = jnp.maximum(send_ref[c, :, :] + comm_ref[idx, c], 0.0)
                xc = jnp.dot(hc.astype(jnp.float32), wout_bf[c * q:(c + 1) * q, :],
                             preferred_element_type=jnp.float32)
                x_f32 = xc if x_f32 is None else x_f32 + xc
            x_cur = x_f32

            if l + 2 < N_LAYERS:
                loads[f"win{l + 2}"] = load_win(l + 2)
                loads[f"wout{l + 2}"] = load_wout(l + 2)
        out_ref[:, :] = x_f32

    return pl.pallas_call(
        body,
        out_shape=jax.ShapeDtypeStruct((b, d_in), jnp.float32),
        in_specs=[pl.BlockSpec(memory_space=pltpu.VMEM)]
        + [pl.BlockSpec(memory_space=pl.ANY)] * 6,
        out_specs=pl.BlockSpec(memory_space=pltpu.VMEM),
        scratch_shapes=[
            pltpu.VMEM((N_CHAINS, b, h_dim // N_CHAINS), jnp.bfloat16),
            pltpu.VMEM((N_LAYERS * N_STEPS, N_CHAINS, b, h_dim // N_CHAINS), jnp.bfloat16),
            pltpu.VMEM((2, d_in, h_dim), jnp.float32),
            pltpu.VMEM((2, h_dim, d_in), jnp.float32),
            pltpu.SemaphoreType.DMA((N_LAYERS * N_STEPS, N_CHAINS)),
            pltpu.SemaphoreType.DMA((N_LAYERS * N_STEPS, N_CHAINS)),
            pltpu.SemaphoreType.DMA((2 * N_LAYERS,)),
        ],
        compiler_params=pltpu.CompilerParams(collective_id=0),
    )(x, Win0, Wout0, Win1, Wout1, Win2, Wout2)
